# baseline (speedup 1.0000x reference)
import functools

import jax
import jax.numpy as jnp
from jax import lax
from jax.experimental import pallas as pl
from jax.experimental.pallas import tpu as pltpu
from jax.experimental.pallas import tpu_sc as plsc

_N, _E, _D, _H, _C, _T = 10000, 320000, 128, 128, 3, 4

_NC, _NS = 2, 16
_NW = _NC * _NS

_CH = 128
_NCH = 80
_E_PAD = _NW * _NCH * _CH

_ROWS = _NS * _CH * (-(-_N // (_NS * _CH)))
_RPS = _ROWS // _NS

_A_PAD = 2048
_APW = _A_PAD // _NW

_MM = dict(preferred_element_type=jnp.float32, precision=lax.Precision.DEFAULT)


_GRP = 8
_NGRP = _NCH // _GRP


def _zero_agg_slice(msg0, agg_sh, base):
    @pl.loop(0, _CH)
    def _(r):
        for k in range(_H // 16):
            msg0[r, pl.ds(k * 16, 16)] = jnp.zeros((16,), jnp.float32)

    for b in range(_RPS // _CH):
        pltpu.sync_copy(msg0, agg_sh.at[pl.ds(base + b * _CH, _CH)])


def _sc_scatter_body(gix_hbm, typ_hbm, dst_hbm, y_hbm, *refs):
    (agg_out, ig0, it0, id0, ig1, it1, id1, msg0, msg1, agg_sh,
     gsem0, gsem1, isem0, isem1) = refs
    c = lax.axis_index("c")
    s = lax.axis_index("s")
    w = c * _NS + s
    base = s * _RPS

    igs, its, ids = (ig0, ig1), (it0, it1), (id0, id1)
    msgs, gsems, isems = (msg0, msg1), (gsem0, gsem1), (isem0, isem1)

    def stage(g, p):
        row0 = w * _NCH + g * _GRP
        pltpu.async_copy(gix_hbm.at[pl.ds(row0, _GRP)], igs[p], isems[p])
        pltpu.async_copy(typ_hbm.at[pl.ds(row0, _GRP)], its[p], isems[p])
        pltpu.async_copy(dst_hbm.at[pl.ds(row0, _GRP)], ids[p], isems[p])

    def stage_wait(p):
        row0 = w * _NCH
        pltpu.make_async_copy(gix_hbm.at[pl.ds(row0, _GRP)], igs[p], isems[p]).wait()
        pltpu.make_async_copy(typ_hbm.at[pl.ds(row0, _GRP)], its[p], isems[p]).wait()
        pltpu.make_async_copy(dst_hbm.at[pl.ds(row0, _GRP)], ids[p], isems[p]).wait()

    def gather(p, j, b):
        pltpu.async_copy(y_hbm.at[igs[p].at[j]], msgs[b], gsems[b])

    def gather_wait(p, j, b):
        pltpu.make_async_copy(y_hbm.at[igs[p].at[j]], msgs[b], gsems[b]).wait()

    _zero_agg_slice(msg0, agg_sh, base)
    plsc.subcore_barrier()

    stage(0, 0)
    stage(1, 1)

    @pl.loop(0, _NGRP // 2)
    def _(u):
        for p in range(2):
            g = u * 2 + p
            stage_wait(p)

            @pl.loop(0, _GRP)
            def _(r):
                for k in range(_CH // 16):
                    sl = pl.ds(k * 16, 16)
                    igs[p][r, sl] = igs[p][r, sl] * _T + its[p][r, sl]

            gather(p, 0, 0)
            gather(p, 1, 1)
            for j in range(_GRP):
                b = j % 2
                gather_wait(p, j, b)
                pltpu.sync_copy(msgs[b], agg_sh.at[ids[p].at[j]], add=True)
                if j + 2 < _GRP:
                    gather(p, j + 2, b)
            gnext = jnp.minimum(g + 2, _NGRP - 2 + p)
            row0 = w * _NCH + gnext * _GRP
            pltpu.async_copy(gix_hbm.at[pl.ds(row0, _GRP)], igs[p], isems[p])
            pltpu.async_copy(typ_hbm.at[pl.ds(row0, _GRP)], its[p], isems[p])
            pltpu.async_copy(dst_hbm.at[pl.ds(row0, _GRP)], ids[p], isems[p])

    stage_wait(0)
    stage_wait(1)

    plsc.subcore_barrier()

    pltpu.sync_copy(agg_sh.at[pl.ds(base, _RPS)],
                    agg_out.at[c, pl.ds(base, _RPS)])


_sc_scatter = pl.kernel(
    _sc_scatter_body,
    out_type=[jax.ShapeDtypeStruct((_NC, _ROWS, _H), jnp.float32)],
    mesh=plsc.VectorSubcoreMesh(core_axis_name="c", subcore_axis_name="s"),
    scratch_types=[
        pltpu.VMEM((_GRP, _CH), jnp.int32),
        pltpu.VMEM((_GRP, _CH), jnp.int32),
        pltpu.VMEM((_GRP, _CH), jnp.int32),
        pltpu.VMEM((_GRP, _CH), jnp.int32),
        pltpu.VMEM((_GRP, _CH), jnp.int32),
        pltpu.VMEM((_GRP, _CH), jnp.int32),
        pltpu.VMEM((_CH, _H), jnp.float32),
        pltpu.VMEM((_CH, _H), jnp.float32),
        pltpu.VMEM_SHARED((_ROWS, _H), jnp.float32),
        pltpu.SemaphoreType.DMA,
        pltpu.SemaphoreType.DMA,
        pltpu.SemaphoreType.DMA,
        pltpu.SemaphoreType.DMA,
    ],
)


def _sc_deg_body(dst_hbm, deg_out, id0, id1, ones_v, agg_sh, ssem, isem0, isem1):
    c = lax.axis_index("c")
    s = lax.axis_index("s")
    w = c * _NS + s
    base = s * _RPS

    ids, isems = (id0, id1), (isem0, isem1)

    _zero_agg_slice(ones_v, agg_sh, base)

    @pl.loop(0, _CH)
    def _(r):
        for k in range(_H // 16):
            ones_v[r, pl.ds(k * 16, 16)] = jnp.ones((16,), jnp.float32)

    plsc.subcore_barrier()

    def stage(g, p):
        row0 = w * _NCH + g * _GRP
        pltpu.async_copy(dst_hbm.at[pl.ds(row0, _GRP)], ids[p], isems[p])

    def stage_wait(p):
        pltpu.make_async_copy(dst_hbm.at[pl.ds(w * _NCH, _GRP)], ids[p],
                              isems[p]).wait()

    def scat_wait():
        pltpu.make_async_copy(ones_v, agg_sh.at[id0.at[0]], ssem).wait()

    stage(0, 0)
    stage(1, 1)

    @pl.loop(0, _NGRP // 2)
    def _(u):
        for p in range(2):
            g = u * 2 + p
            stage_wait(p)
            for j in range(_GRP):
                pltpu.async_copy(ones_v, agg_sh.at[ids[p].at[j]], ssem,
                                 add=True)
            for j in range(_GRP):
                scat_wait()
            gnext = jnp.minimum(g + 2, _NGRP - 2 + p)
            row0 = w * _NCH + gnext * _GRP
            pltpu.async_copy(dst_hbm.at[pl.ds(row0, _GRP)], ids[p], isems[p])

    stage_wait(0)
    stage_wait(1)

    plsc.subcore_barrier()

    pltpu.sync_copy(agg_sh.at[pl.ds(base, _RPS)],
                    deg_out.at[c, pl.ds(base, _RPS)])


_sc_deg = pl.kernel(
    _sc_deg_body,
    out_type=[jax.ShapeDtypeStruct((_NC, _ROWS, _H), jnp.float32)],
    mesh=plsc.VectorSubcoreMesh(core_axis_name="c", subcore_axis_name="s"),
    scratch_types=[
        pltpu.VMEM((_GRP, _CH), jnp.int32),
        pltpu.VMEM((_GRP, _CH), jnp.int32),
        pltpu.VMEM((_CH, _H), jnp.float32),
        pltpu.VMEM_SHARED((_ROWS, _H), jnp.float32),
        pltpu.SemaphoreType.DMA,
        pltpu.SemaphoreType.DMA,
        pltpu.SemaphoreType.DMA,
    ],
)


def _sc_gather_body(aidx_hbm, agg2_hbm, deg_hbm, s2_hbm,
                    ga_out, gd_out, gs_out, idx0, idx1, rows_v, sem0):
    c = lax.axis_index("c")
    s = lax.axis_index("s")
    w = c * _NS + s
    base = w * _APW

    pltpu.sync_copy(aidx_hbm.at[pl.ds(base, _APW)], idx0)

    @pl.loop(0, _APW // 16)
    def _(k):
        sl = pl.ds(k * 16, 16)
        idx1[sl] = idx0[sl] + _ROWS

    pltpu.async_copy(agg2_hbm.at[idx0], rows_v, sem0).wait()
    pltpu.sync_copy(rows_v, ga_out.at[0, pl.ds(base, _APW)])
    pltpu.async_copy(agg2_hbm.at[idx1], rows_v, sem0).wait()
    pltpu.sync_copy(rows_v, ga_out.at[1, pl.ds(base, _APW)])
    pltpu.async_copy(deg_hbm.at[idx0], rows_v, sem0).wait()
    pltpu.sync_copy(rows_v, gd_out.at[0, pl.ds(base, _APW)])
    pltpu.async_copy(deg_hbm.at[idx1], rows_v, sem0).wait()
    pltpu.sync_copy(rows_v, gd_out.at[1, pl.ds(base, _APW)])
    pltpu.async_copy(s2_hbm.at[idx0], rows_v, sem0).wait()
    pltpu.sync_copy(rows_v, gs_out.at[pl.ds(base, _APW)])


_sc_gather = pl.kernel(
    _sc_gather_body,
    out_type=[
        jax.ShapeDtypeStruct((_NC, _A_PAD, _H), jnp.float32),
        jax.ShapeDtypeStruct((_NC, _A_PAD, _H), jnp.float32),
        jax.ShapeDtypeStruct((_A_PAD, _H), jnp.float32),
    ],
    mesh=plsc.VectorSubcoreMesh(core_axis_name="c", subcore_axis_name="s"),
    scratch_types=[
        pltpu.VMEM((_APW,), jnp.int32),
        pltpu.VMEM((_APW,), jnp.int32),
        pltpu.VMEM((_APW, _H), jnp.float32),
        pltpu.SemaphoreType.DMA,
    ],
)


_BN = 128


def _dense1_kernel(x_ref, w_ref, ws_ref, b_ref, y_ref, s_ref):
    x = x_ref[...]
    for t in range(_T):
        y_ref[:, t * _H:(t + 1) * _H] = jnp.dot(x, w_ref[t], **_MM)
    s_ref[...] = jnp.dot(x, ws_ref[...], **_MM) + b_ref[...]


def _combine_kernel(agg_ref, deg_ref, s1_ref, w_ref, ws_ref, b_ref,
                    y_ref, s_ref):
    d = deg_ref[0, :, 0:1] + deg_ref[1, :, 0:1]
    inv = 1.0 / jnp.maximum(d, 1.0)
    h = (agg_ref[0] + agg_ref[1]) * inv + s1_ref[...]
    h = jnp.maximum(h, 0.0)
    for t in range(_T):
        y_ref[:, t * _H:(t + 1) * _H] = jnp.dot(h, w_ref[t], **_MM)
    s_ref[...] = jnp.dot(h, ws_ref[...], **_MM) + b_ref[...]


def _classify_kernel(ga_ref, gd_ref, gs_ref, wc_ref, bc_ref, out_ref):
    d = gd_ref[0, :, 0:1] + gd_ref[1, :, 0:1]
    inv = 1.0 / jnp.maximum(d, 1.0)
    h2 = (ga_ref[0] + ga_ref[1]) * inv + gs_ref[...]
    out_ref[...] = jnp.dot(h2, wc_ref[...], **_MM) + bc_ref[...]


def _dense1(x, W, Ws, b):
    grid = (_ROWS // _BN,)
    return pl.pallas_call(
        _dense1_kernel,
        grid=grid,
        in_specs=[
            pl.BlockSpec((_BN, _D), lambda i: (i, 0)),
            pl.BlockSpec((_T, _D, _H), lambda i: (0, 0, 0)),
            pl.BlockSpec((_D, _H), lambda i: (0, 0)),
            pl.BlockSpec((1, _H), lambda i: (0, 0)),
        ],
        out_specs=[
            pl.BlockSpec((_BN, _T * _H), lambda i: (i, 0)),
            pl.BlockSpec((_BN, _H), lambda i: (i, 0)),
        ],
        out_shape=[
            jax.ShapeDtypeStruct((_ROWS, _T * _H), jnp.float32),
            jax.ShapeDtypeStruct((_ROWS, _H), jnp.float32),
        ],
    )(x, W, Ws, b)


def _combine(agg, deg, s1, W, Ws, b):
    grid = (_ROWS // _BN,)
    return pl.pallas_call(
        _combine_kernel,
        grid=grid,
        in_specs=[
            pl.BlockSpec((_NC, _BN, _H), lambda i: (0, i, 0)),
            pl.BlockSpec((_NC, _BN, _H), lambda i: (0, i, 0)),
            pl.BlockSpec((_BN, _H), lambda i: (i, 0)),
            pl.BlockSpec((_T, _D, _H), lambda i: (0, 0, 0)),
            pl.BlockSpec((_D, _H), lambda i: (0, 0)),
            pl.BlockSpec((1, _H), lambda i: (0, 0)),
        ],
        out_specs=[
            pl.BlockSpec((_BN, _T * _H), lambda i: (i, 0)),
            pl.BlockSpec((_BN, _H), lambda i: (i, 0)),
        ],
        out_shape=[
            jax.ShapeDtypeStruct((_ROWS, _T * _H), jnp.float32),
            jax.ShapeDtypeStruct((_ROWS, _H), jnp.float32),
        ],
    )(agg, deg, s1, W, Ws, b)


def _classify(ga, gd, gs, Wc, bc):
    return pl.pallas_call(
        _classify_kernel,
        in_specs=[
            pl.BlockSpec((_NC, _A_PAD, _H), lambda: (0, 0, 0)),
            pl.BlockSpec((_NC, _A_PAD, _H), lambda: (0, 0, 0)),
            pl.BlockSpec((_A_PAD, _H), lambda: (0, 0)),
            pl.BlockSpec((_D, _C), lambda: (0, 0)),
            pl.BlockSpec((1, _C), lambda: (0, 0)),
        ],
        out_specs=pl.BlockSpec((_A_PAD, _C), lambda: (0, 0)),
        out_shape=jax.ShapeDtypeStruct((_A_PAD, _C), jnp.float32),
    )(ga, gd, gs, Wc, bc)


@jax.jit
def kernel(features, edge_index, edge_types, aspect_indices,
           W1, W1s, b1, W2, W2s, b2, Wc, bc):
    src = edge_index[0].astype(jnp.int32)
    dst = edge_index[1].astype(jnp.int32)
    typ = edge_types.astype(jnp.int32)

    pad = _E_PAD - _E
    src_p = jnp.concatenate([src, jnp.zeros((pad,), jnp.int32)])
    typ_p = jnp.concatenate([typ, jnp.zeros((pad,), jnp.int32)])
    dump = _N + jnp.arange(pad, dtype=jnp.int32) % (_ROWS - _N)
    dst_p = jnp.concatenate([dst, dump])
    src2d = src_p.reshape(_E_PAD // _CH, _CH)
    typ2d = typ_p.reshape(_E_PAD // _CH, _CH)
    dst2d = dst_p.reshape(_E_PAD // _CH, _CH)

    apad = jnp.concatenate(
        [aspect_indices.astype(jnp.int32),
         jnp.zeros((_A_PAD - aspect_indices.shape[0],), jnp.int32)])

    x = jnp.pad(features, ((0, _ROWS - _N), (0, 0)))

    deg, = _sc_deg(dst2d)

    y1, s1 = _dense1(x, W1, W1s, b1.reshape(1, _H))
    y1f = y1.reshape(_ROWS * _T, _H)
    agg1, = _sc_scatter(src2d, typ2d, dst2d, y1f)

    y2, s2 = _combine(agg1, deg, s1, W2, W2s, b2.reshape(1, _H))
    y2f = y2.reshape(_ROWS * _T, _H)

    agg2, = _sc_scatter(src2d, typ2d, dst2d, y2f)

    agg2f = agg2.reshape(_NC * _ROWS, _H)
    degf = deg.reshape(_NC * _ROWS, _H)
    ga, gd, gs = _sc_gather(apad, agg2f, degf, s2)
    logits = _classify(ga, gd, gs, Wc, bc.reshape(1, _C))
    return logits[:aspect_indices.shape[0]]


# speedup vs baseline: 1.0882x; 1.0882x over previous
import functools

import jax
import jax.numpy as jnp
from jax import lax
from jax.experimental import pallas as pl
from jax.experimental.pallas import tpu as pltpu
from jax.experimental.pallas import tpu_sc as plsc

_N, _E, _D, _H, _C, _T = 10000, 320000, 128, 128, 3, 4

_NC, _NS = 2, 16
_NW = _NC * _NS

_CH = 128
_NCH = 80
_E_PAD = _NW * _NCH * _CH

_ROWS = _NS * _CH * (-(-_N // (_NS * _CH)))
_RPS = _ROWS // _NS

_A_PAD = 2048
_APW = _A_PAD // _NW

_MM = dict(preferred_element_type=jnp.float32, precision=lax.Precision.DEFAULT)


_GRP = 8
_NGRP = _NCH // _GRP


def _zero_agg_slice(msg0, agg_sh, base):
    @pl.loop(0, _CH)
    def _(r):
        for k in range(_H // 16):
            msg0[r, pl.ds(k * 16, 16)] = jnp.zeros((16,), jnp.float32)

    for b in range(_RPS // _CH):
        pltpu.sync_copy(msg0, agg_sh.at[pl.ds(base + b * _CH, _CH)])


def _sc_scatter_body(gix_hbm, typ_hbm, dst_hbm, y_hbm, *refs):
    (agg_out, ig0, it0, id0, ig1, it1, id1, msg0, msg1, agg_sh,
     gsem0, gsem1, isem0, isem1) = refs
    c = lax.axis_index("c")
    s = lax.axis_index("s")
    w = c * _NS + s
    base = s * _RPS

    igs, its, ids = (ig0, ig1), (it0, it1), (id0, id1)
    msgs, gsems, isems = (msg0, msg1), (gsem0, gsem1), (isem0, isem1)

    def stage(g, p):
        row0 = w * _NCH + g * _GRP
        pltpu.async_copy(gix_hbm.at[pl.ds(row0, _GRP)], igs[p], isems[p])
        pltpu.async_copy(typ_hbm.at[pl.ds(row0, _GRP)], its[p], isems[p])
        pltpu.async_copy(dst_hbm.at[pl.ds(row0, _GRP)], ids[p], isems[p])

    def stage_wait(p):
        row0 = w * _NCH
        pltpu.make_async_copy(gix_hbm.at[pl.ds(row0, _GRP)], igs[p], isems[p]).wait()
        pltpu.make_async_copy(typ_hbm.at[pl.ds(row0, _GRP)], its[p], isems[p]).wait()
        pltpu.make_async_copy(dst_hbm.at[pl.ds(row0, _GRP)], ids[p], isems[p]).wait()

    def gather(p, j, b):
        pltpu.async_copy(y_hbm.at[igs[p].at[j]], msgs[b], gsems[b])

    def gather_wait(p, j, b):
        pltpu.make_async_copy(y_hbm.at[igs[p].at[j]], msgs[b], gsems[b]).wait()

    _zero_agg_slice(msg0, agg_sh, base)
    plsc.subcore_barrier()

    stage(0, 0)
    stage(1, 1)

    @pl.loop(0, _NGRP // 2)
    def _(u):
        for p in range(2):
            g = u * 2 + p
            stage_wait(p)

            @pl.loop(0, _GRP)
            def _(r):
                for k in range(_CH // 16):
                    sl = pl.ds(k * 16, 16)
                    igs[p][r, sl] = igs[p][r, sl] * _T + its[p][r, sl]

            gather(p, 0, 0)
            gather(p, 1, 1)
            for j in range(_GRP):
                b = j % 2
                gather_wait(p, j, b)
                pltpu.sync_copy(msgs[b], agg_sh.at[ids[p].at[j]], add=True)
                if j + 2 < _GRP:
                    gather(p, j + 2, b)
            gnext = jnp.minimum(g + 2, _NGRP - 2 + p)
            row0 = w * _NCH + gnext * _GRP
            pltpu.async_copy(gix_hbm.at[pl.ds(row0, _GRP)], igs[p], isems[p])
            pltpu.async_copy(typ_hbm.at[pl.ds(row0, _GRP)], its[p], isems[p])
            pltpu.async_copy(dst_hbm.at[pl.ds(row0, _GRP)], ids[p], isems[p])

    stage_wait(0)
    stage_wait(1)

    plsc.subcore_barrier()

    pltpu.sync_copy(agg_sh.at[pl.ds(base, _RPS)],
                    agg_out.at[c, pl.ds(base, _RPS)])


_sc_scatter = pl.kernel(
    _sc_scatter_body,
    out_type=[jax.ShapeDtypeStruct((_NC, _ROWS, _H), jnp.float32)],
    mesh=plsc.VectorSubcoreMesh(core_axis_name="c", subcore_axis_name="s"),
    scratch_types=[
        pltpu.VMEM((_GRP, _CH), jnp.int32),
        pltpu.VMEM((_GRP, _CH), jnp.int32),
        pltpu.VMEM((_GRP, _CH), jnp.int32),
        pltpu.VMEM((_GRP, _CH), jnp.int32),
        pltpu.VMEM((_GRP, _CH), jnp.int32),
        pltpu.VMEM((_GRP, _CH), jnp.int32),
        pltpu.VMEM((_CH, _H), jnp.float32),
        pltpu.VMEM((_CH, _H), jnp.float32),
        pltpu.VMEM_SHARED((_ROWS, _H), jnp.float32),
        pltpu.SemaphoreType.DMA,
        pltpu.SemaphoreType.DMA,
        pltpu.SemaphoreType.DMA,
        pltpu.SemaphoreType.DMA,
    ],
)


def _sc_deg_body(dst_hbm, deg_out, id0, id1, ones_v, agg_sh, ssem, isem0, isem1):
    c = lax.axis_index("c")
    s = lax.axis_index("s")
    w = c * _NS + s
    base = s * _RPS

    ids, isems = (id0, id1), (isem0, isem1)

    _zero_agg_slice(ones_v, agg_sh, base)

    @pl.loop(0, _CH)
    def _(r):
        for k in range(_H // 16):
            ones_v[r, pl.ds(k * 16, 16)] = jnp.ones((16,), jnp.float32)

    plsc.subcore_barrier()

    def stage(g, p):
        row0 = w * _NCH + g * _GRP
        pltpu.async_copy(dst_hbm.at[pl.ds(row0, _GRP)], ids[p], isems[p])

    def stage_wait(p):
        pltpu.make_async_copy(dst_hbm.at[pl.ds(w * _NCH, _GRP)], ids[p],
                              isems[p]).wait()

    def scat_wait():
        pltpu.make_async_copy(ones_v, agg_sh.at[id0.at[0]], ssem).wait()

    stage(0, 0)
    stage(1, 1)

    @pl.loop(0, _NGRP // 2)
    def _(u):
        for p in range(2):
            g = u * 2 + p
            stage_wait(p)
            for j in range(_GRP):
                pltpu.async_copy(ones_v, agg_sh.at[ids[p].at[j]], ssem,
                                 add=True)
            for j in range(_GRP):
                scat_wait()
            gnext = jnp.minimum(g + 2, _NGRP - 2 + p)
            row0 = w * _NCH + gnext * _GRP
            pltpu.async_copy(dst_hbm.at[pl.ds(row0, _GRP)], ids[p], isems[p])

    stage_wait(0)
    stage_wait(1)

    plsc.subcore_barrier()

    pltpu.sync_copy(agg_sh.at[pl.ds(base, _RPS)],
                    deg_out.at[c, pl.ds(base, _RPS)])


_sc_deg = pl.kernel(
    _sc_deg_body,
    out_type=[jax.ShapeDtypeStruct((_NC, _ROWS, _H), jnp.float32)],
    mesh=plsc.VectorSubcoreMesh(core_axis_name="c", subcore_axis_name="s"),
    scratch_types=[
        pltpu.VMEM((_GRP, _CH), jnp.int32),
        pltpu.VMEM((_GRP, _CH), jnp.int32),
        pltpu.VMEM((_CH, _H), jnp.float32),
        pltpu.VMEM_SHARED((_ROWS, _H), jnp.float32),
        pltpu.SemaphoreType.DMA,
        pltpu.SemaphoreType.DMA,
        pltpu.SemaphoreType.DMA,
    ],
)


def _sc_gather_body(aidx_hbm, h2_hbm, gh_out, idx0, rows_v, sem0):
    c = lax.axis_index("c")
    s = lax.axis_index("s")
    w = c * _NS + s
    base = w * _APW

    pltpu.sync_copy(aidx_hbm.at[pl.ds(base, _APW)], idx0)
    pltpu.async_copy(h2_hbm.at[idx0], rows_v, sem0).wait()
    pltpu.sync_copy(rows_v, gh_out.at[pl.ds(base, _APW)])


_sc_gather = pl.kernel(
    _sc_gather_body,
    out_type=[jax.ShapeDtypeStruct((_A_PAD, _H), jnp.float32)],
    mesh=plsc.VectorSubcoreMesh(core_axis_name="c", subcore_axis_name="s"),
    scratch_types=[
        pltpu.VMEM((_APW,), jnp.int32),
        pltpu.VMEM((_APW, _H), jnp.float32),
        pltpu.SemaphoreType.DMA,
    ],
)


_BN = 128


def _dense1_kernel(x_ref, w_ref, ws_ref, b_ref, y_ref, s_ref):
    x = x_ref[...]
    for t in range(_T):
        y_ref[:, t * _H:(t + 1) * _H] = jnp.dot(x, w_ref[t], **_MM)
    s_ref[...] = jnp.dot(x, ws_ref[...], **_MM) + b_ref[...]


def _combine_kernel(agg_ref, deg_ref, s1_ref, w_ref, ws_ref, b_ref,
                    y_ref, s_ref):
    d = deg_ref[0, :, 0:1] + deg_ref[1, :, 0:1]
    inv = 1.0 / jnp.maximum(d, 1.0)
    h = (agg_ref[0] + agg_ref[1]) * inv + s1_ref[...]
    h = jnp.maximum(h, 0.0)
    for t in range(_T):
        y_ref[:, t * _H:(t + 1) * _H] = jnp.dot(h, w_ref[t], **_MM)
    s_ref[...] = jnp.dot(h, ws_ref[...], **_MM) + b_ref[...]


def _combineb_kernel(agg_ref, deg_ref, s2_ref, h_ref):
    d = deg_ref[0, :, 0:1] + deg_ref[1, :, 0:1]
    inv = 1.0 / jnp.maximum(d, 1.0)
    h_ref[...] = (agg_ref[0] + agg_ref[1]) * inv + s2_ref[...]


def _classify_kernel(gh_ref, wc_ref, bc_ref, out_ref):
    out_ref[...] = jnp.dot(gh_ref[...], wc_ref[...], **_MM) + bc_ref[...]


def _dense1(x, W, Ws, b):
    grid = (_ROWS // _BN,)
    return pl.pallas_call(
        _dense1_kernel,
        grid=grid,
        in_specs=[
            pl.BlockSpec((_BN, _D), lambda i: (i, 0)),
            pl.BlockSpec((_T, _D, _H), lambda i: (0, 0, 0)),
            pl.BlockSpec((_D, _H), lambda i: (0, 0)),
            pl.BlockSpec((1, _H), lambda i: (0, 0)),
        ],
        out_specs=[
            pl.BlockSpec((_BN, _T * _H), lambda i: (i, 0)),
            pl.BlockSpec((_BN, _H), lambda i: (i, 0)),
        ],
        out_shape=[
            jax.ShapeDtypeStruct((_ROWS, _T * _H), jnp.float32),
            jax.ShapeDtypeStruct((_ROWS, _H), jnp.float32),
        ],
    )(x, W, Ws, b)


def _combine(agg, deg, s1, W, Ws, b):
    grid = (_ROWS // _BN,)
    return pl.pallas_call(
        _combine_kernel,
        grid=grid,
        in_specs=[
            pl.BlockSpec((_NC, _BN, _H), lambda i: (0, i, 0)),
            pl.BlockSpec((_NC, _BN, _H), lambda i: (0, i, 0)),
            pl.BlockSpec((_BN, _H), lambda i: (i, 0)),
            pl.BlockSpec((_T, _D, _H), lambda i: (0, 0, 0)),
            pl.BlockSpec((_D, _H), lambda i: (0, 0)),
            pl.BlockSpec((1, _H), lambda i: (0, 0)),
        ],
        out_specs=[
            pl.BlockSpec((_BN, _T * _H), lambda i: (i, 0)),
            pl.BlockSpec((_BN, _H), lambda i: (i, 0)),
        ],
        out_shape=[
            jax.ShapeDtypeStruct((_ROWS, _T * _H), jnp.float32),
            jax.ShapeDtypeStruct((_ROWS, _H), jnp.float32),
        ],
    )(agg, deg, s1, W, Ws, b)


def _combineb(agg, deg, s2):
    grid = (_ROWS // _BN,)
    return pl.pallas_call(
        _combineb_kernel,
        grid=grid,
        in_specs=[
            pl.BlockSpec((_NC, _BN, _H), lambda i: (0, i, 0)),
            pl.BlockSpec((_NC, _BN, _H), lambda i: (0, i, 0)),
            pl.BlockSpec((_BN, _H), lambda i: (i, 0)),
        ],
        out_specs=pl.BlockSpec((_BN, _H), lambda i: (i, 0)),
        out_shape=jax.ShapeDtypeStruct((_ROWS, _H), jnp.float32),
    )(agg, deg, s2)


def _classify(gh, Wc, bc):
    return pl.pallas_call(
        _classify_kernel,
        in_specs=[
            pl.BlockSpec((_A_PAD, _H), lambda: (0, 0)),
            pl.BlockSpec((_D, _C), lambda: (0, 0)),
            pl.BlockSpec((1, _C), lambda: (0, 0)),
        ],
        out_specs=pl.BlockSpec((_A_PAD, _C), lambda: (0, 0)),
        out_shape=jax.ShapeDtypeStruct((_A_PAD, _C), jnp.float32),
    )(gh, Wc, bc)


@jax.jit
def kernel(features, edge_index, edge_types, aspect_indices,
           W1, W1s, b1, W2, W2s, b2, Wc, bc):
    src = edge_index[0].astype(jnp.int32)
    dst = edge_index[1].astype(jnp.int32)
    typ = edge_types.astype(jnp.int32)

    pad = _E_PAD - _E
    src_p = jnp.concatenate([src, jnp.zeros((pad,), jnp.int32)])
    typ_p = jnp.concatenate([typ, jnp.zeros((pad,), jnp.int32)])
    dump = _N + jnp.arange(pad, dtype=jnp.int32) % (_ROWS - _N)
    dst_p = jnp.concatenate([dst, dump])
    src2d = src_p.reshape(_E_PAD // _CH, _CH)
    typ2d = typ_p.reshape(_E_PAD // _CH, _CH)
    dst2d = dst_p.reshape(_E_PAD // _CH, _CH)

    apad = jnp.concatenate(
        [aspect_indices.astype(jnp.int32),
         jnp.zeros((_A_PAD - aspect_indices.shape[0],), jnp.int32)])

    x = jnp.pad(features, ((0, _ROWS - _N), (0, 0)))

    deg, = _sc_deg(dst2d)

    y1, s1 = _dense1(x, W1, W1s, b1.reshape(1, _H))
    y1f = y1.reshape(_ROWS * _T, _H)
    agg1, = _sc_scatter(src2d, typ2d, dst2d, y1f)

    y2, s2 = _combine(agg1, deg, s1, W2, W2s, b2.reshape(1, _H))
    y2f = y2.reshape(_ROWS * _T, _H)

    agg2, = _sc_scatter(src2d, typ2d, dst2d, y2f)

    h2 = _combineb(agg2, deg, s2)
    gh, = _sc_gather(apad, h2)
    logits = _classify(gh, Wc, bc.reshape(1, _C))
    return logits[:aspect_indices.shape[0]]


# speedup vs baseline: 1.1525x; 1.0591x over previous
import jax
import jax.numpy as jnp
from jax import lax
from jax.experimental import pallas as pl
from jax.experimental.pallas import tpu as pltpu
from jax.experimental.pallas import tpu_sc as plsc

_N, _E, _D, _H, _C, _T = 10000, 320000, 128, 128, 3, 4

_NC, _NS = 2, 16
_NW = _NC * _NS

_CH = 128
_NCH = 80
_E_PAD = _NW * _NCH * _CH

_ROWS = _NS * _CH * (-(-_N // (_NS * _CH)))
_RPS = _ROWS // _NS

_A_PAD = 2048
_APW = _A_PAD // _NW

_MM = dict(preferred_element_type=jnp.float32, precision=lax.Precision.DEFAULT)


_GRP = 8
_NGRP = _NCH // _GRP
_NCH0 = 128
_NCH1 = 2 * _NCH - _NCH0
_SC0_CHUNKS = _NS * _NCH0


def _zero_agg_slice(msg0, agg_sh, base):
    @pl.loop(0, _CH)
    def _(r):
        for k in range(_H // 16):
            msg0[r, pl.ds(k * 16, 16)] = jnp.zeros((16,), jnp.float32)

    for b in range(_RPS // _CH):
        pltpu.sync_copy(msg0, agg_sh.at[pl.ds(base + b * _CH, _CH)])


def _sc_scatter_body(gix_hbm, typ_hbm, dst_hbm, y_hbm, *refs):
    (agg_out, ig0, it0, id0, ig1, it1, id1, msg0, msg1, agg_sh,
     gsem0, gsem1, isem0, isem1) = refs
    c = lax.axis_index("c")
    s = lax.axis_index("s")
    base = s * _RPS

    nch = jnp.where(c == 0, _NCH0, _NCH1)
    ch0 = jnp.where(c == 0, s * _NCH0, _SC0_CHUNKS + s * _NCH1)
    ngrp = nch // _GRP

    igs, its, ids = (ig0, ig1), (it0, it1), (id0, id1)
    msgs, gsems, isems = (msg0, msg1), (gsem0, gsem1), (isem0, isem1)

    def stage(g, p):
        row0 = ch0 + g * _GRP
        pltpu.async_copy(gix_hbm.at[pl.ds(row0, _GRP)], igs[p], isems[p])
        pltpu.async_copy(typ_hbm.at[pl.ds(row0, _GRP)], its[p], isems[p])
        pltpu.async_copy(dst_hbm.at[pl.ds(row0, _GRP)], ids[p], isems[p])

    def stage_wait(p):
        pltpu.make_async_copy(gix_hbm.at[pl.ds(ch0, _GRP)], igs[p], isems[p]).wait()
        pltpu.make_async_copy(typ_hbm.at[pl.ds(ch0, _GRP)], its[p], isems[p]).wait()
        pltpu.make_async_copy(dst_hbm.at[pl.ds(ch0, _GRP)], ids[p], isems[p]).wait()

    def gather(p, j, b):
        pltpu.async_copy(y_hbm.at[igs[p].at[j]], msgs[b], gsems[b])

    def gather_wait(p, j, b):
        pltpu.make_async_copy(y_hbm.at[igs[p].at[j]], msgs[b], gsems[b]).wait()

    _zero_agg_slice(msg0, agg_sh, base)
    plsc.subcore_barrier()

    stage(0, 0)
    stage(1, 1)

    @pl.loop(0, ngrp // 2)
    def _(u):
        for p in range(2):
            g = u * 2 + p
            stage_wait(p)

            @pl.loop(0, _GRP)
            def _(r):
                for k in range(_CH // 16):
                    sl = pl.ds(k * 16, 16)
                    igs[p][r, sl] = igs[p][r, sl] * _T + its[p][r, sl]

            gather(p, 0, 0)
            gather(p, 1, 1)
            for j in range(_GRP):
                b = j % 2
                gather_wait(p, j, b)
                pltpu.sync_copy(msgs[b], agg_sh.at[ids[p].at[j]], add=True)
                if j + 2 < _GRP:
                    gather(p, j + 2, b)
            gnext = jnp.minimum(g + 2, ngrp - 2 + p)
            row0 = ch0 + gnext * _GRP
            pltpu.async_copy(gix_hbm.at[pl.ds(row0, _GRP)], igs[p], isems[p])
            pltpu.async_copy(typ_hbm.at[pl.ds(row0, _GRP)], its[p], isems[p])
            pltpu.async_copy(dst_hbm.at[pl.ds(row0, _GRP)], ids[p], isems[p])

    stage_wait(0)
    stage_wait(1)

    plsc.subcore_barrier()

    pltpu.sync_copy(agg_sh.at[pl.ds(base, _RPS)],
                    agg_out.at[c, pl.ds(base, _RPS)])


_sc_scatter = pl.kernel(
    _sc_scatter_body,
    out_type=[jax.ShapeDtypeStruct((_NC, _ROWS, _H), jnp.float32)],
    mesh=plsc.VectorSubcoreMesh(core_axis_name="c", subcore_axis_name="s"),
    scratch_types=[
        pltpu.VMEM((_GRP, _CH), jnp.int32),
        pltpu.VMEM((_GRP, _CH), jnp.int32),
        pltpu.VMEM((_GRP, _CH), jnp.int32),
        pltpu.VMEM((_GRP, _CH), jnp.int32),
        pltpu.VMEM((_GRP, _CH), jnp.int32),
        pltpu.VMEM((_GRP, _CH), jnp.int32),
        pltpu.VMEM((_CH, _H), jnp.float32),
        pltpu.VMEM((_CH, _H), jnp.float32),
        pltpu.VMEM_SHARED((_ROWS, _H), jnp.float32),
        pltpu.SemaphoreType.DMA,
        pltpu.SemaphoreType.DMA,
        pltpu.SemaphoreType.DMA,
        pltpu.SemaphoreType.DMA,
    ],
)


def _sc_deg_body(dst_hbm, deg_out, id0, id1, ones_v, agg_sh, ssem, isem0, isem1):
    c = lax.axis_index("c")
    s = lax.axis_index("s")
    w = c * _NS + s
    base = s * _RPS

    ids, isems = (id0, id1), (isem0, isem1)

    _zero_agg_slice(ones_v, agg_sh, base)

    @pl.loop(0, _CH)
    def _(r):
        for k in range(_H // 16):
            ones_v[r, pl.ds(k * 16, 16)] = jnp.ones((16,), jnp.float32)

    plsc.subcore_barrier()

    def stage(g, p):
        row0 = w * _NCH + g * _GRP
        pltpu.async_copy(dst_hbm.at[pl.ds(row0, _GRP)], ids[p], isems[p])

    def stage_wait(p):
        pltpu.make_async_copy(dst_hbm.at[pl.ds(w * _NCH, _GRP)], ids[p],
                              isems[p]).wait()

    def scat_wait():
        pltpu.make_async_copy(ones_v, agg_sh.at[id0.at[0]], ssem).wait()

    stage(0, 0)
    stage(1, 1)

    @pl.loop(0, _NGRP // 2)
    def _(u):
        for p in range(2):
            g = u * 2 + p
            stage_wait(p)
            for j in range(_GRP):
                pltpu.async_copy(ones_v, agg_sh.at[ids[p].at[j]], ssem,
                                 add=True)
            for j in range(_GRP):
                scat_wait()
            gnext = jnp.minimum(g + 2, _NGRP - 2 + p)
            row0 = w * _NCH + gnext * _GRP
            pltpu.async_copy(dst_hbm.at[pl.ds(row0, _GRP)], ids[p], isems[p])

    stage_wait(0)
    stage_wait(1)

    plsc.subcore_barrier()

    pltpu.sync_copy(agg_sh.at[pl.ds(base, _RPS)],
                    deg_out.at[c, pl.ds(base, _RPS)])


_sc_deg = pl.kernel(
    _sc_deg_body,
    out_type=[jax.ShapeDtypeStruct((_NC, _ROWS, _H), jnp.float32)],
    mesh=plsc.VectorSubcoreMesh(core_axis_name="c", subcore_axis_name="s"),
    scratch_types=[
        pltpu.VMEM((_GRP, _CH), jnp.int32),
        pltpu.VMEM((_GRP, _CH), jnp.int32),
        pltpu.VMEM((_CH, _H), jnp.float32),
        pltpu.VMEM_SHARED((_ROWS, _H), jnp.float32),
        pltpu.SemaphoreType.DMA,
        pltpu.SemaphoreType.DMA,
        pltpu.SemaphoreType.DMA,
    ],
)


def _sc_gather_body(aidx_hbm, h2_hbm, gh_out, idx0, rows_v, sem0):
    c = lax.axis_index("c")
    s = lax.axis_index("s")
    w = c * _NS + s
    base = w * _APW

    pltpu.sync_copy(aidx_hbm.at[pl.ds(base, _APW)], idx0)
    pltpu.async_copy(h2_hbm.at[idx0], rows_v, sem0).wait()
    pltpu.sync_copy(rows_v, gh_out.at[pl.ds(base, _APW)])


_sc_gather = pl.kernel(
    _sc_gather_body,
    out_type=[jax.ShapeDtypeStruct((_A_PAD, _H), jnp.float32)],
    mesh=plsc.VectorSubcoreMesh(core_axis_name="c", subcore_axis_name="s"),
    scratch_types=[
        pltpu.VMEM((_APW,), jnp.int32),
        pltpu.VMEM((_APW, _H), jnp.float32),
        pltpu.SemaphoreType.DMA,
    ],
)


_BN = 128


def _dense1_kernel(x_ref, w_ref, ws_ref, b_ref, y_ref, s_ref):
    x = x_ref[...]
    for t in range(_T):
        y_ref[:, t * _H:(t + 1) * _H] = jnp.dot(x, w_ref[t], **_MM)
    s_ref[...] = jnp.dot(x, ws_ref[...], **_MM) + b_ref[...]


def _combine_kernel(agg_ref, deg_ref, s1_ref, w_ref, ws_ref, b_ref,
                    y_ref, s_ref):
    d = deg_ref[0, :, 0:1] + deg_ref[1, :, 0:1]
    inv = 1.0 / jnp.maximum(d, 1.0)
    h = (agg_ref[0] + agg_ref[1]) * inv + s1_ref[...]
    h = jnp.maximum(h, 0.0)
    for t in range(_T):
        y_ref[:, t * _H:(t + 1) * _H] = jnp.dot(h, w_ref[t], **_MM)
    s_ref[...] = jnp.dot(h, ws_ref[...], **_MM) + b_ref[...]


def _combineb_kernel(agg_ref, deg_ref, s2_ref, h_ref):
    d = deg_ref[0, :, 0:1] + deg_ref[1, :, 0:1]
    inv = 1.0 / jnp.maximum(d, 1.0)
    h_ref[...] = (agg_ref[0] + agg_ref[1]) * inv + s2_ref[...]


def _classify_kernel(gh_ref, wc_ref, bc_ref, out_ref):
    out_ref[...] = jnp.dot(gh_ref[...], wc_ref[...], **_MM) + bc_ref[...]


def _dense1(x, W, Ws, b):
    grid = (_ROWS // _BN,)
    return pl.pallas_call(
        _dense1_kernel,
        grid=grid,
        in_specs=[
            pl.BlockSpec((_BN, _D), lambda i: (i, 0)),
            pl.BlockSpec((_T, _D, _H), lambda i: (0, 0, 0)),
            pl.BlockSpec((_D, _H), lambda i: (0, 0)),
            pl.BlockSpec((1, _H), lambda i: (0, 0)),
        ],
        out_specs=[
            pl.BlockSpec((_BN, _T * _H), lambda i: (i, 0)),
            pl.BlockSpec((_BN, _H), lambda i: (i, 0)),
        ],
        out_shape=[
            jax.ShapeDtypeStruct((_ROWS, _T * _H), jnp.float32),
            jax.ShapeDtypeStruct((_ROWS, _H), jnp.float32),
        ],
    )(x, W, Ws, b)


def _combine(agg, deg, s1, W, Ws, b):
    grid = (_ROWS // _BN,)
    return pl.pallas_call(
        _combine_kernel,
        grid=grid,
        in_specs=[
            pl.BlockSpec((_NC, _BN, _H), lambda i: (0, i, 0)),
            pl.BlockSpec((_NC, _BN, _H), lambda i: (0, i, 0)),
            pl.BlockSpec((_BN, _H), lambda i: (i, 0)),
            pl.BlockSpec((_T, _D, _H), lambda i: (0, 0, 0)),
            pl.BlockSpec((_D, _H), lambda i: (0, 0)),
            pl.BlockSpec((1, _H), lambda i: (0, 0)),
        ],
        out_specs=[
            pl.BlockSpec((_BN, _T * _H), lambda i: (i, 0)),
            pl.BlockSpec((_BN, _H), lambda i: (i, 0)),
        ],
        out_shape=[
            jax.ShapeDtypeStruct((_ROWS, _T * _H), jnp.float32),
            jax.ShapeDtypeStruct((_ROWS, _H), jnp.float32),
        ],
    )(agg, deg, s1, W, Ws, b)


def _combineb(agg, deg, s2):
    grid = (_ROWS // _BN,)
    return pl.pallas_call(
        _combineb_kernel,
        grid=grid,
        in_specs=[
            pl.BlockSpec((_NC, _BN, _H), lambda i: (0, i, 0)),
            pl.BlockSpec((_NC, _BN, _H), lambda i: (0, i, 0)),
            pl.BlockSpec((_BN, _H), lambda i: (i, 0)),
        ],
        out_specs=pl.BlockSpec((_BN, _H), lambda i: (i, 0)),
        out_shape=jax.ShapeDtypeStruct((_ROWS, _H), jnp.float32),
    )(agg, deg, s2)


def _classify(gh, Wc, bc):
    return pl.pallas_call(
        _classify_kernel,
        in_specs=[
            pl.BlockSpec((_A_PAD, _H), lambda: (0, 0)),
            pl.BlockSpec((_D, _C), lambda: (0, 0)),
            pl.BlockSpec((1, _C), lambda: (0, 0)),
        ],
        out_specs=pl.BlockSpec((_A_PAD, _C), lambda: (0, 0)),
        out_shape=jax.ShapeDtypeStruct((_A_PAD, _C), jnp.float32),
    )(gh, Wc, bc)


@jax.jit
def kernel(features, edge_index, edge_types, aspect_indices,
           W1, W1s, b1, W2, W2s, b2, Wc, bc):
    src = edge_index[0].astype(jnp.int32)
    dst = edge_index[1].astype(jnp.int32)
    typ = edge_types.astype(jnp.int32)

    pad = _E_PAD - _E
    src_p = jnp.concatenate([src, jnp.zeros((pad,), jnp.int32)])
    typ_p = jnp.concatenate([typ, jnp.zeros((pad,), jnp.int32)])
    dump = _N + jnp.arange(pad, dtype=jnp.int32) % (_ROWS - _N)
    dst_p = jnp.concatenate([dst, dump])
    src2d = src_p.reshape(_E_PAD // _CH, _CH)
    typ2d = typ_p.reshape(_E_PAD // _CH, _CH)
    dst2d = dst_p.reshape(_E_PAD // _CH, _CH)

    apad = jnp.concatenate(
        [aspect_indices.astype(jnp.int32),
         jnp.zeros((_A_PAD - aspect_indices.shape[0],), jnp.int32)])

    x = jnp.pad(features, ((0, _ROWS - _N), (0, 0)))

    deg, = _sc_deg(dst2d)

    y1, s1 = _dense1(x, W1, W1s, b1.reshape(1, _H))
    y1f = y1.reshape(_ROWS * _T, _H)
    agg1, = _sc_scatter(src2d, typ2d, dst2d, y1f)

    y2, s2 = _combine(agg1, deg, s1, W2, W2s, b2.reshape(1, _H))
    y2f = y2.reshape(_ROWS * _T, _H)

    agg2, = _sc_scatter(src2d, typ2d, dst2d, y2f)

    h2 = _combineb(agg2, deg, s2)
    gh, = _sc_gather(apad, h2)
    logits = _classify(gh, Wc, bc.reshape(1, _C))
    return logits[:aspect_indices.shape[0]]
